# Initial kernel scaffold; baseline (speedup 1.0000x reference)
import jax
import jax.numpy as jnp
from jax.experimental import pallas as pl


def kernel(f, p, W_sc, b_sc, W_pre, b_pre, W_p2f, b_p2f, W_mlp, b_mlp, W_pst, b_pst):
    raise NotImplementedError("write your pallas kernel here")


# speedup vs baseline: 1.3890x; 1.3890x over previous
import functools
import jax
import jax.numpy as jnp
from jax.experimental import pallas as pl
from jax.experimental.pallas import tpu as pltpu

B, N, D_IN, D_OUT, D_HID = 4, 4096, 128, 256, 64
K = 32


def _prologue_body(f_ref, p_ref, wpre_ref, bpre_ref, wsc_ref, bsc_ref,
                   wpab_ref, bp2f_ref, t_ref, c_ref, res_ref):
    f = f_ref[0]
    p = p_ref[0]
    h = f @ wpre_ref[...] + bpre_ref[...]
    h = jnp.where(h > 0, h, 0.01 * h)
    pa = p @ wpab_ref[...]
    t_ref[0] = h + pa[:, D_HID:]
    c_ref[0] = pa[:, :D_HID] - h + bp2f_ref[...]
    res_ref[0] = f @ wsc_ref[...] + bsc_ref[...]


def _prologue(f, p, W_pre, b_pre, W_sc, b_sc, W_pab, b_p2f):
    T = 1024
    grid = (B, N // T)
    return pl.pallas_call(
        _prologue_body,
        grid=grid,
        in_specs=[
            pl.BlockSpec((1, T, D_IN), lambda b, i: (b, i, 0)),
            pl.BlockSpec((1, T, 3), lambda b, i: (b, i, 0)),
            pl.BlockSpec((D_IN, D_HID), lambda b, i: (0, 0)),
            pl.BlockSpec((D_HID,), lambda b, i: (0,)),
            pl.BlockSpec((D_IN, D_OUT), lambda b, i: (0, 0)),
            pl.BlockSpec((D_OUT,), lambda b, i: (0,)),
            pl.BlockSpec((3, 2 * D_HID), lambda b, i: (0, 0)),
            pl.BlockSpec((D_HID,), lambda b, i: (0,)),
        ],
        out_specs=[
            pl.BlockSpec((1, T, D_HID), lambda b, i: (b, i, 0)),
            pl.BlockSpec((1, T, D_HID), lambda b, i: (b, i, 0)),
            pl.BlockSpec((1, T, D_OUT), lambda b, i: (b, i, 0)),
        ],
        out_shape=[
            jax.ShapeDtypeStruct((B, N, D_HID), jnp.float32),
            jax.ShapeDtypeStruct((B, N, D_HID), jnp.float32),
            jax.ShapeDtypeStruct((B, N, D_OUT), jnp.float32),
        ],
    )(f, p, W_pre, b_pre, W_sc, b_sc, W_pab, b_p2f)


def _tail_body(tg_ref, c_ref, res_ref, wmlp_ref, bmlp_ref, wpst_ref,
               bpst_ref, out_ref):
    Trows = c_ref.shape[1]
    tg = tg_ref[0]
    c = c_ref[0]
    g = tg + jnp.repeat(c, K, axis=0)
    g = jnp.where(g > 0, g, 0.01 * g)
    z = g @ wmlp_ref[...] + bmlp_ref[...]
    z = jnp.where(z > 0, z, 0.01 * z)
    m = jnp.max(z.reshape(Trows, K, D_HID), axis=1)
    o = m @ wpst_ref[...] + bpst_ref[...] + res_ref[0]
    out_ref[0] = jnp.where(o > 0, o, 0.01 * o)


def _tail(tg, c, res, W_mlp, b_mlp, W_pst, b_pst):
    T = 512
    grid = (B, N // T)
    return pl.pallas_call(
        _tail_body,
        grid=grid,
        in_specs=[
            pl.BlockSpec((1, T * K, D_HID), lambda b, i: (b, i, 0)),
            pl.BlockSpec((1, T, D_HID), lambda b, i: (b, i, 0)),
            pl.BlockSpec((1, T, D_OUT), lambda b, i: (b, i, 0)),
            pl.BlockSpec((D_HID, D_HID), lambda b, i: (0, 0)),
            pl.BlockSpec((D_HID,), lambda b, i: (0,)),
            pl.BlockSpec((D_HID, D_OUT), lambda b, i: (0, 0)),
            pl.BlockSpec((D_OUT,), lambda b, i: (0,)),
        ],
        out_specs=pl.BlockSpec((1, T, D_OUT), lambda b, i: (b, i, 0)),
        out_shape=jax.ShapeDtypeStruct((B, N, D_OUT), jnp.float32),
    )(tg, c, res, W_mlp, b_mlp, W_pst, b_pst)


def kernel(f, p, W_sc, b_sc, W_pre, b_pre, W_p2f, b_p2f, W_mlp, b_mlp,
           W_pst, b_pst):
    A = W_p2f[0:3] - W_p2f[6:9]
    Bm = W_p2f[3:6] + W_p2f[6:9]
    W_pab = jnp.concatenate([A, Bm], axis=1)

    t, c, res = _prologue(f, p, W_pre, b_pre, W_sc, b_sc, W_pab, b_p2f)

    sq = jnp.sum(p * p, axis=-1)
    d2 = sq[:, :, None] - 2.0 * jnp.einsum('bnd,bmd->bnm', p, p) + sq[:, None, :]
    _, idx = jax.lax.top_k(-d2, K)

    tg = jax.vmap(lambda tb, ib: tb[ib])(t, idx)
    tg = tg.reshape(B, N * K, D_HID)

    out = _tail(tg, c, res, W_mlp, b_mlp, W_pst, b_pst)
    return (out, p)


# speedup vs baseline: 12.1303x; 8.7328x over previous
import functools
import jax
import jax.numpy as jnp
from jax.experimental import pallas as pl
from jax.experimental.pallas import tpu as pltpu

B, N, D_IN, D_OUT, D_HID = 4, 4096, 128, 256, 64
K = 32


def _prologue_body(f_ref, p_ref, wpre_ref, bpre_ref, wsc_ref, bsc_ref,
                   wpab_ref, bp2f_ref, t_ref, c_ref, res_ref):
    f = f_ref[0]
    p = p_ref[0]
    h = f @ wpre_ref[...] + bpre_ref[...]
    h = jnp.where(h > 0, h, 0.01 * h)
    pa = p @ wpab_ref[...]
    t_ref[0] = h + pa[:, D_HID:]
    c_ref[0] = pa[:, :D_HID] - h + bp2f_ref[...]
    res_ref[0] = f @ wsc_ref[...] + bsc_ref[...]


def _prologue(f, p, W_pre, b_pre, W_sc, b_sc, W_pab, b_p2f):
    T = 1024
    grid = (B, N // T)
    return pl.pallas_call(
        _prologue_body,
        grid=grid,
        in_specs=[
            pl.BlockSpec((1, T, D_IN), lambda b, i: (b, i, 0)),
            pl.BlockSpec((1, T, 3), lambda b, i: (b, i, 0)),
            pl.BlockSpec((D_IN, D_HID), lambda b, i: (0, 0)),
            pl.BlockSpec((D_HID,), lambda b, i: (0,)),
            pl.BlockSpec((D_IN, D_OUT), lambda b, i: (0, 0)),
            pl.BlockSpec((D_OUT,), lambda b, i: (0,)),
            pl.BlockSpec((3, 2 * D_HID), lambda b, i: (0, 0)),
            pl.BlockSpec((D_HID,), lambda b, i: (0,)),
        ],
        out_specs=[
            pl.BlockSpec((1, T, D_HID), lambda b, i: (b, i, 0)),
            pl.BlockSpec((1, T, D_HID), lambda b, i: (b, i, 0)),
            pl.BlockSpec((1, T, D_OUT), lambda b, i: (b, i, 0)),
        ],
        out_shape=[
            jax.ShapeDtypeStruct((B, N, D_HID), jnp.float32),
            jax.ShapeDtypeStruct((B, N, D_HID), jnp.float32),
            jax.ShapeDtypeStruct((B, N, D_OUT), jnp.float32),
        ],
    )(f, p, W_pre, b_pre, W_sc, b_sc, W_pab, b_p2f)


def _tail_body(tg_ref, c_ref, res_ref, wmlp_ref, bmlp_ref, wpst_ref,
               bpst_ref, out_ref):
    Trows = c_ref.shape[1]
    tg = tg_ref[0]
    c = c_ref[0]
    g = tg + c[None, :, :]
    g = jnp.where(g > 0, g, 0.01 * g)
    z = g.reshape(K * Trows, D_HID) @ wmlp_ref[...] + bmlp_ref[...]
    z = jnp.where(z > 0, z, 0.01 * z)
    m = jnp.max(z.reshape(K, Trows, D_HID), axis=0)
    o = m @ wpst_ref[...] + bpst_ref[...] + res_ref[0]
    out_ref[0] = jnp.where(o > 0, o, 0.01 * o)


def _tail(tg, c, res, W_mlp, b_mlp, W_pst, b_pst):
    T = 512
    grid = (B, N // T)
    return pl.pallas_call(
        _tail_body,
        grid=grid,
        in_specs=[
            pl.BlockSpec((1, K, T, D_HID), lambda b, i: (b, 0, i, 0)),
            pl.BlockSpec((1, T, D_HID), lambda b, i: (b, i, 0)),
            pl.BlockSpec((1, T, D_OUT), lambda b, i: (b, i, 0)),
            pl.BlockSpec((D_HID, D_HID), lambda b, i: (0, 0)),
            pl.BlockSpec((D_HID,), lambda b, i: (0,)),
            pl.BlockSpec((D_HID, D_OUT), lambda b, i: (0, 0)),
            pl.BlockSpec((D_OUT,), lambda b, i: (0,)),
        ],
        out_specs=pl.BlockSpec((1, T, D_OUT), lambda b, i: (b, i, 0)),
        out_shape=jax.ShapeDtypeStruct((B, N, D_OUT), jnp.float32),
    )(tg, c, res, W_mlp, b_mlp, W_pst, b_pst)


def _cmpex(ka, xa, kb, xb):
    m = kb < ka
    klo = jnp.where(m, kb, ka)
    khi = jnp.where(m, ka, kb)
    xlo = jnp.where(m, xb, xa)
    xhi = jnp.where(m, xa, xb)
    return klo, xlo, khi, xhi


def _bitonic_clean(k, x):
    R = k.shape[0]
    d = R // 2
    while d >= 1:
        kparts, xparts = [], []
        for b in range(0, R, 2 * d):
            klo, xlo, khi, xhi = _cmpex(k[b:b + d], x[b:b + d],
                                        k[b + d:b + 2 * d], x[b + d:b + 2 * d])
            kparts += [klo, khi]
            xparts += [xlo, xhi]
        k = jnp.concatenate(kparts, axis=0)
        x = jnp.concatenate(xparts, axis=0)
        d //= 2
    return k, x


def _merge_pairs(k, x, truncate):
    R, L, _ = k.shape
    ka, xa = k[:, :L // 2], x[:, :L // 2]

    def _rev(a):
        if a.shape[0] == 1:
            return a
        return jnp.concatenate([a[i:i + 1] for i in range(a.shape[0] - 1, -1, -1)],
                               axis=0)
    kb = _rev(k[:, L // 2:])
    xb = _rev(x[:, L // 2:])
    if truncate:
        klo, xlo, _, _ = _cmpex(ka, xa, kb, xb)
        return _bitonic_clean(klo, xlo)
    kc = jnp.concatenate([ka, kb], axis=0)
    xc = jnp.concatenate([xa, xb], axis=0)
    return _bitonic_clean(kc, xc)


def _knn_body(pfull_ref, ptile_ref, idx_ref):
    b = pl.program_id(0)
    p_all = pfull_ref[0]
    p_til = ptile_ref[0]
    T = p_til.shape[0]
    sq = jnp.sum(p_all * p_all, axis=1)
    dot = jax.lax.dot_general(p_all, p_til, (((1,), (1,)), ((), ())),
                              preferred_element_type=jnp.float32)
    key = sq[:, None] - 2.0 * dot
    x = jax.lax.broadcasted_iota(jnp.int32, (N, T), 0) + b * N

    k = key.reshape(1, N, T)
    x = x.reshape(1, N, T)
    while k.shape[0] < K:
        k, x = _merge_pairs(k, x, truncate=False)
    while k.shape[1] > 1:
        k, x = _merge_pairs(k, x, truncate=True)
    idx_ref[0] = x[:, 0, :]


def _knn(p):
    T = 128
    grid = (B, N // T)
    return pl.pallas_call(
        _knn_body,
        grid=grid,
        in_specs=[
            pl.BlockSpec((1, N, 3), lambda b, i: (b, 0, 0)),
            pl.BlockSpec((1, T, 3), lambda b, i: (b, i, 0)),
        ],
        out_specs=pl.BlockSpec((1, K, T), lambda b, i: (b, 0, i)),
        out_shape=jax.ShapeDtypeStruct((B, K, N), jnp.int32),
    )(p, p)


def kernel(f, p, W_sc, b_sc, W_pre, b_pre, W_p2f, b_p2f, W_mlp, b_mlp,
           W_pst, b_pst):
    A = W_p2f[0:3] - W_p2f[6:9]
    Bm = W_p2f[3:6] + W_p2f[6:9]
    W_pab = jnp.concatenate([A, Bm], axis=1)

    t, c, res = _prologue(f, p, W_pre, b_pre, W_sc, b_sc, W_pab, b_p2f)

    idx = _knn(p)

    tg = jnp.take(t.reshape(B * N, D_HID), idx.reshape(-1), axis=0)
    tg = tg.reshape(B, K, N, D_HID)

    out = _tail(tg, c, res, W_mlp, b_mlp, W_pst, b_pst)
    return (out, p)


# speedup vs baseline: 30.5824x; 2.5212x over previous
import functools
import jax
import jax.numpy as jnp
from jax.experimental import pallas as pl
from jax.experimental.pallas import tpu as pltpu
from jax.experimental.pallas import tpu_sc as plsc

B, N, D_IN, D_OUT, D_HID = 4, 4096, 128, 256, 64
K = 32


def _prologue_body(f_ref, p_ref, wpre_ref, bpre_ref, wsc_ref, bsc_ref,
                   wpab_ref, bp2f_ref, tc_ref, res_ref):
    f = f_ref[0]
    p = p_ref[0]
    h = f @ wpre_ref[...] + bpre_ref[...]
    h = jnp.maximum(h, 0.01 * h)
    pa = p @ wpab_ref[...]
    t = h + pa[:, D_HID:]
    c = pa[:, :D_HID] - h + bp2f_ref[...]
    tc_ref[0] = jnp.concatenate([t, c], axis=1)
    res_ref[0] = f @ wsc_ref[...] + bsc_ref[...]


def _prologue(f, p, W_pre, b_pre, W_sc, b_sc, W_pab, b_p2f):
    T = 1024
    grid = (B, N // T)
    return pl.pallas_call(
        _prologue_body,
        grid=grid,
        in_specs=[
            pl.BlockSpec((1, T, D_IN), lambda b, i: (b, i, 0)),
            pl.BlockSpec((1, T, 3), lambda b, i: (b, i, 0)),
            pl.BlockSpec((D_IN, D_HID), lambda b, i: (0, 0)),
            pl.BlockSpec((D_HID,), lambda b, i: (0,)),
            pl.BlockSpec((D_IN, D_OUT), lambda b, i: (0, 0)),
            pl.BlockSpec((D_OUT,), lambda b, i: (0,)),
            pl.BlockSpec((3, 2 * D_HID), lambda b, i: (0, 0)),
            pl.BlockSpec((D_HID,), lambda b, i: (0,)),
        ],
        out_specs=[
            pl.BlockSpec((1, T, 2 * D_HID), lambda b, i: (b, i, 0)),
            pl.BlockSpec((1, T, D_OUT), lambda b, i: (b, i, 0)),
        ],
        out_shape=[
            jax.ShapeDtypeStruct((B, N, 2 * D_HID), jnp.float32),
            jax.ShapeDtypeStruct((B, N, D_OUT), jnp.float32),
        ],
    )(f, p, W_pre, b_pre, W_sc, b_sc, W_pab, b_p2f)


def _tail_body(tg_ref, c_ref, res_ref, wmlp_ref, bmlp_ref, wpst_ref,
               bpst_ref, out_ref):
    Trows = c_ref.shape[1]
    tg = tg_ref[0, :, :, :D_HID]
    c = c_ref[0, :, D_HID:]
    g = tg + c[None, :, :]
    g = jnp.maximum(g, 0.01 * g)
    z = g.reshape(K * Trows, D_HID) @ wmlp_ref[...] + bmlp_ref[...]
    z = jnp.maximum(z, 0.01 * z)
    m = jnp.max(z.reshape(K, Trows, D_HID), axis=0)
    o = m @ wpst_ref[...] + bpst_ref[...] + res_ref[0]
    out_ref[0] = jnp.maximum(o, 0.01 * o)


def _tail(tg, tc_pack, res, W_mlp, b_mlp, W_pst, b_pst):
    T = 512
    grid = (B, N // T)
    return pl.pallas_call(
        _tail_body,
        grid=grid,
        in_specs=[
            pl.BlockSpec((1, K, T, 2 * D_HID), lambda b, i: (b, 0, i, 0)),
            pl.BlockSpec((1, T, 2 * D_HID), lambda b, i: (b, i, 0)),
            pl.BlockSpec((1, T, D_OUT), lambda b, i: (b, i, 0)),
            pl.BlockSpec((D_HID, D_HID), lambda b, i: (0, 0)),
            pl.BlockSpec((D_HID,), lambda b, i: (0,)),
            pl.BlockSpec((D_HID, D_OUT), lambda b, i: (0, 0)),
            pl.BlockSpec((D_OUT,), lambda b, i: (0,)),
        ],
        out_specs=pl.BlockSpec((1, T, D_OUT), lambda b, i: (b, i, 0)),
        out_shape=jax.ShapeDtypeStruct((B, N, D_OUT), jnp.float32),
    )(tg, tc_pack, res, W_mlp, b_mlp, W_pst, b_pst)


def _cmpex(ka, xa, kb, xb):
    m = kb < ka
    klo = jnp.where(m, kb, ka)
    khi = jnp.where(m, ka, kb)
    xlo = jnp.where(m, xb, xa)
    xhi = jnp.where(m, xa, xb)
    return klo, xlo, khi, xhi


def _bitonic_clean(k, x):
    R = k.shape[0]
    d = R // 2
    while d >= 1:
        kparts, xparts = [], []
        for b in range(0, R, 2 * d):
            klo, xlo, khi, xhi = _cmpex(k[b:b + d], x[b:b + d],
                                        k[b + d:b + 2 * d], x[b + d:b + 2 * d])
            kparts += [klo, khi]
            xparts += [xlo, xhi]
        k = jnp.concatenate(kparts, axis=0)
        x = jnp.concatenate(xparts, axis=0)
        d //= 2
    return k, x


def _merge_pairs(k, x, truncate):
    R, L, _ = k.shape
    ka, xa = k[:, :L // 2], x[:, :L // 2]

    def _rev(a):
        if a.shape[0] == 1:
            return a
        return jnp.concatenate([a[i:i + 1] for i in range(a.shape[0] - 1, -1, -1)],
                               axis=0)
    kb = _rev(k[:, L // 2:])
    xb = _rev(x[:, L // 2:])
    if truncate:
        klo, xlo, _, _ = _cmpex(ka, xa, kb, xb)
        return _bitonic_clean(klo, xlo)
    kc = jnp.concatenate([ka, kb], axis=0)
    xc = jnp.concatenate([xa, xb], axis=0)
    return _bitonic_clean(kc, xc)


def _knn_body(pfull_ref, ptile_ref, idx_ref):
    b = pl.program_id(0)
    p_all = pfull_ref[0]
    p_til = ptile_ref[0]
    T = p_til.shape[0]
    sq = jnp.sum(p_all * p_all, axis=1)
    dot = jax.lax.dot_general(p_all, p_til, (((1,), (1,)), ((), ())),
                              preferred_element_type=jnp.float32)
    key = sq[:, None] - 2.0 * dot
    x = jax.lax.broadcasted_iota(jnp.int32, (N, T), 0) + b * N

    k = key.reshape(1, N, T)
    x = x.reshape(1, N, T)
    while k.shape[0] < K:
        k, x = _merge_pairs(k, x, truncate=False)
    while k.shape[1] > 1:
        k, x = _merge_pairs(k, x, truncate=True)
    idx_ref[0] = x[:, 0, :]


def _knn(p):
    T = 128
    grid = (B, N // T)
    return pl.pallas_call(
        _knn_body,
        grid=grid,
        in_specs=[
            pl.BlockSpec((1, N, 3), lambda b, i: (b, 0, 0)),
            pl.BlockSpec((1, T, 3), lambda b, i: (b, i, 0)),
        ],
        out_specs=pl.BlockSpec((1, K, T), lambda b, i: (b, 0, i)),
        out_shape=jax.ShapeDtypeStruct((B, K, N), jnp.int32),
    )(p, p)


_N_IDX = B * K * N
_SC_W = 32
_PER_W = _N_IDX // _SC_W
_CHUNK = 512
_ROW_W = 2 * D_HID


def _sc_gather_body(t_ref, idx_ref, out_ref, idx_v, rows_v, sem):
    wid = jax.lax.axis_index("c") * 16 + jax.lax.axis_index("s")
    base = wid * _PER_W

    def body(i, carry):
        off = base + i * _CHUNK
        pltpu.sync_copy(idx_ref.at[pl.ds(off, _CHUNK)], idx_v)
        pltpu.async_copy(t_ref.at[idx_v], rows_v, sem).wait()
        pltpu.sync_copy(rows_v, out_ref.at[pl.ds(off, _CHUNK)])
        return carry

    jax.lax.fori_loop(0, _PER_W // _CHUNK, body, 0)


def _sc_gather(t_flat, idx_flat):
    mesh = plsc.VectorSubcoreMesh(core_axis_name="c", subcore_axis_name="s")
    fn = functools.partial(
        pl.kernel,
        mesh=mesh,
        out_type=jax.ShapeDtypeStruct((_N_IDX, _ROW_W), jnp.float32),
        scratch_types=[
            pltpu.VMEM((_CHUNK,), jnp.int32),
            pltpu.VMEM((_CHUNK, _ROW_W), jnp.float32),
            pltpu.SemaphoreType.DMA,
        ],
    )(_sc_gather_body)
    return fn(t_flat, idx_flat)


def kernel(f, p, W_sc, b_sc, W_pre, b_pre, W_p2f, b_p2f, W_mlp, b_mlp,
           W_pst, b_pst):
    A = W_p2f[0:3] - W_p2f[6:9]
    Bm = W_p2f[3:6] + W_p2f[6:9]
    W_pab = jnp.concatenate([A, Bm], axis=1)

    tc_pack, res = _prologue(f, p, W_pre, b_pre, W_sc, b_sc, W_pab, b_p2f)

    idx = _knn(p)

    tg = _sc_gather(tc_pack.reshape(B * N, _ROW_W), idx.reshape(-1))
    tg = tg.reshape(B, K, N, _ROW_W)

    out = _tail(tg, tc_pack, res, W_mlp, b_mlp, W_pst, b_pst)
    return (out, p)
